# baseline (speedup 1.0000x reference)
import functools

import numpy as np
import jax
import jax.numpy as jnp
from jax import lax
from jax.experimental import pallas as pl
from jax.experimental.pallas import tpu as pltpu
from jax.experimental.pallas import tpu_sc as plsc

_B, _E, _N, _D = 16, 16384, 1024, 128
_ET = 4096
_LOGEPS = float(np.log(np.float32(1e-10)))

_NC, _NS = 2, 16
_NW = _NC * _NS
_CELLS = _B * _N * _N
_CH = 128
_EPT = _B * _E // _NW
_NCH = _EPT // _CH
_EROWS = _B * _E // _CH


def _mlp_body(x_ref, ei_ref, w0_ref, b0_ref, w1_ref, b1_ref, wo_ref, bo_ref,
              val_ref, idx_ref, heat_ref):
    b = pl.program_id(0)
    x = x_ref[0]
    dn = (((1,), (1,)), ((), ()))
    h = lax.dot_general(x, w0_ref[...], dn, preferred_element_type=jnp.float32)
    h = jax.nn.silu(h + b0_ref[0])
    h = lax.dot_general(h, w1_ref[...], dn, preferred_element_type=jnp.float32)
    h = jax.nn.silu(h + b1_ref[0])
    z = lax.dot_general(wo_ref[...], h, dn,
                        preferred_element_type=jnp.float32) + bo_ref[0, 0]
    e = jax.nn.sigmoid(z)
    val_ref[...] = jnp.log(e + 1e-10)[:, None, :]
    src = ei_ref[0, 0:1]
    dst = ei_ref[0, 1:2]
    idx_ref[...] = (b * (_N * _N) + src * _N + dst)[:, None, :]
    heat_ref[...] = jnp.full(heat_ref.shape, _LOGEPS, jnp.float32)


def _run_mlp(edge_attr, edge_index, W0, b0, W1, b1, Wout, bout):
    grid = (_B, _E // _ET)
    vals, idx, heat = pl.pallas_call(
        _mlp_body,
        grid=grid,
        in_specs=[
            pl.BlockSpec((1, _ET, _D), lambda b, j: (b, j, 0)),
            pl.BlockSpec((1, 2, _ET), lambda b, j: (b, 0, j)),
            pl.BlockSpec((_D, _D), lambda b, j: (0, 0)),
            pl.BlockSpec((1, _D), lambda b, j: (0, 0)),
            pl.BlockSpec((_D, _D), lambda b, j: (0, 0)),
            pl.BlockSpec((1, _D), lambda b, j: (0, 0)),
            pl.BlockSpec((1, _D), lambda b, j: (0, 0)),
            pl.BlockSpec((1, 1), lambda b, j: (0, 0)),
        ],
        out_specs=[
            pl.BlockSpec((1, 1, _ET), lambda b, j: (b * (_E // _ET) + j, 0, 0)),
            pl.BlockSpec((1, 1, _ET), lambda b, j: (b * (_E // _ET) + j, 0, 0)),
            pl.BlockSpec((1, _N // (_E // _ET), _N), lambda b, j: (b, j, 0)),
        ],
        out_shape=[
            jax.ShapeDtypeStruct((_B * _E // _ET, 1, _ET), jnp.float32),
            jax.ShapeDtypeStruct((_B * _E // _ET, 1, _ET), jnp.int32),
            jax.ShapeDtypeStruct((_B, _N, _N), jnp.float32),
        ],
    )(edge_attr, edge_index, W0, b0.reshape(1, _D), W1, b1.reshape(1, _D),
      Wout.reshape(1, _D), bout.reshape(1, 1))
    return vals, idx, heat


_sc_mesh = plsc.VectorSubcoreMesh(core_axis_name="c", subcore_axis_name="s")


@functools.partial(
    pl.kernel,
    mesh=_sc_mesh,
    scratch_types=[
        pltpu.VMEM((_NCH, _CH), jnp.int32),
        pltpu.VMEM((_NCH, _CH), jnp.float32),
        pltpu.SemaphoreType.DMA,
    ],
)
def _sc_scatter(idx_hbm, val_hbm, out_hbm, idx_v, val_v, sem_s):
    c = lax.axis_index("c")
    s = lax.axis_index("s")
    w = c * _NS + s

    rb = w * _NCH
    pltpu.make_async_copy(idx_hbm.at[pl.ds(rb, _NCH)], idx_v, sem_s).start()
    pltpu.make_async_copy(val_hbm.at[pl.ds(rb, _NCH)], val_v, sem_s).start()
    pltpu.make_async_copy(idx_hbm.at[pl.ds(rb, _NCH)], idx_v, sem_s).wait()
    pltpu.make_async_copy(val_hbm.at[pl.ds(rb, _NCH)], val_v, sem_s).wait()

    def fire_scat(j, carry):
        pltpu.make_async_copy(
            val_v.at[j], out_hbm.at[idx_v.at[j]], sem_s).start()
        return carry

    lax.fori_loop(0, _NCH, fire_scat, 0)

    def drain_scat(j, carry):
        pltpu.make_async_copy(
            val_v.at[j], out_hbm.at[idx_v.at[j]], sem_s).wait()
        return carry

    lax.fori_loop(0, _NCH, drain_scat, 0)


def kernel(edge_attr, edge_index, num_nodes, W0, b0, W1, b1, Wout, bout):
    del num_nodes
    ei = edge_index.astype(jnp.int32)
    vals, idx, heat = _run_mlp(edge_attr, ei, W0, b0, W1, b1, Wout, bout)
    idx2 = idx.reshape(_EROWS, _CH)
    vals2 = vals.reshape(_EROWS, _CH)
    heat_ref = jax.new_ref(heat.reshape(_CELLS))
    _sc_scatter(idx2, vals2, heat_ref)
    return heat_ref[...].reshape(_B, _N, _N)


# speedup vs baseline: 1.1236x; 1.1236x over previous
import functools

import numpy as np
import jax
import jax.numpy as jnp
from jax import lax
from jax.experimental import pallas as pl
from jax.experimental.pallas import tpu as pltpu
from jax.experimental.pallas import tpu_sc as plsc

_B, _E, _N, _D = 16, 16384, 1024, 128
_ET = 4096
_LOGEPS = float(np.log(np.float32(1e-10)))

_NC, _NS = 2, 16
_NW = _NC * _NS
_CELLS = _B * _N * _N
_CH = 128
_EPT = _B * _E // _NW
_NCH = _EPT // _CH
_EROWS = _B * _E // _CH


def _mlp_body(x_ref, ei_ref, w0_ref, b0_ref, w1_ref, b1_ref, wo_ref, bo_ref,
              val_ref, idx_ref, heat_ref):
    b = pl.program_id(0)
    x = x_ref[0]
    dn = (((1,), (1,)), ((), ()))
    h = lax.dot_general(x, w0_ref[...], dn, preferred_element_type=jnp.float32)
    h = jax.nn.silu(h + b0_ref[0])
    h = lax.dot_general(h, w1_ref[...], dn, preferred_element_type=jnp.float32)
    h = jax.nn.silu(h + b1_ref[0])
    z = lax.dot_general(wo_ref[...], h, dn,
                        preferred_element_type=jnp.float32) + bo_ref[0, 0]
    e = jax.nn.sigmoid(z)
    val_ref[...] = jnp.log(e + 1e-10)[:, None, :]
    src = ei_ref[0, 0:1]
    dst = ei_ref[0, 1:2]
    idx_ref[...] = (b * (_N * _N) + src * _N + dst)[:, None, :]
    heat_ref[...] = jnp.full(heat_ref.shape, _LOGEPS, jnp.float32)


_HBLK = _CELLS // (_B * _E // _ET)


def _run_mlp(edge_attr, edge_index, W0, b0, W1, b1, Wout, bout):
    grid = (_B, _E // _ET)
    vals, idx, heat = pl.pallas_call(
        _mlp_body,
        grid=grid,
        in_specs=[
            pl.BlockSpec((1, _ET, _D), lambda b, j: (b, j, 0)),
            pl.BlockSpec((1, 2, _ET), lambda b, j: (b, 0, j)),
            pl.BlockSpec((_D, _D), lambda b, j: (0, 0)),
            pl.BlockSpec((1, _D), lambda b, j: (0, 0)),
            pl.BlockSpec((_D, _D), lambda b, j: (0, 0)),
            pl.BlockSpec((1, _D), lambda b, j: (0, 0)),
            pl.BlockSpec((1, _D), lambda b, j: (0, 0)),
            pl.BlockSpec((1, 1), lambda b, j: (0, 0)),
        ],
        out_specs=[
            pl.BlockSpec((1, 1, _ET), lambda b, j: (b * (_E // _ET) + j, 0, 0)),
            pl.BlockSpec((1, 1, _ET), lambda b, j: (b * (_E // _ET) + j, 0, 0)),
            pl.BlockSpec((_HBLK,), lambda b, j: (b * (_E // _ET) + j,)),
        ],
        out_shape=[
            jax.ShapeDtypeStruct((_B * _E // _ET, 1, _ET), jnp.float32),
            jax.ShapeDtypeStruct((_B * _E // _ET, 1, _ET), jnp.int32),
            jax.ShapeDtypeStruct((_CELLS,), jnp.float32),
        ],
    )(edge_attr, edge_index, W0, b0.reshape(1, _D), W1, b1.reshape(1, _D),
      Wout.reshape(1, _D), bout.reshape(1, 1))
    return vals, idx, heat


_sc_mesh = plsc.VectorSubcoreMesh(core_axis_name="c", subcore_axis_name="s")


@functools.partial(
    pl.kernel,
    mesh=_sc_mesh,
    scratch_types=[
        pltpu.VMEM((_EPT,), jnp.int32),
        pltpu.VMEM((_EPT,), jnp.float32),
        pltpu.SemaphoreType.DMA,
    ],
)
def _sc_scatter(idx_hbm, val_hbm, out_hbm, idx_v, val_v, sem_s):
    c = lax.axis_index("c")
    s = lax.axis_index("s")
    w = c * _NS + s

    rb = w * _EPT
    pltpu.make_async_copy(idx_hbm.at[pl.ds(rb, _EPT)], idx_v, sem_s).start()
    pltpu.make_async_copy(val_hbm.at[pl.ds(rb, _EPT)], val_v, sem_s).start()
    pltpu.make_async_copy(idx_hbm.at[pl.ds(rb, _EPT)], idx_v, sem_s).wait()
    pltpu.make_async_copy(val_hbm.at[pl.ds(rb, _EPT)], val_v, sem_s).wait()

    pltpu.make_async_copy(val_v, out_hbm.at[idx_v], sem_s).start()
    pltpu.make_async_copy(val_v, out_hbm.at[idx_v], sem_s).wait()


def kernel(edge_attr, edge_index, num_nodes, W0, b0, W1, b1, Wout, bout):
    del num_nodes
    ei = edge_index.astype(jnp.int32)
    vals, idx, heat = _run_mlp(edge_attr, ei, W0, b0, W1, b1, Wout, bout)
    idx2 = idx.reshape(_B * _E)
    vals2 = vals.reshape(_B * _E)
    heat_ref = jax.new_ref(heat)
    _sc_scatter(idx2, vals2, heat_ref)
    return heat_ref[...].reshape(_B, _N, _N)
